# baseline (speedup 1.0000x reference)
import jax
import jax.numpy as jnp
from jax.experimental import pallas as pl

_EMB = 32
_NE = 8192
_NTOK = 8192
_TB = 256
_NBLK = _NTOK // _TB


def _vq_body(x_ref, w_ref, xsq_ref, wsq_ref, idx_ref, qst_ref, cnt_ref, dw_ref,
             loss_ref, ppl_ref):
    i = pl.program_id(0)
    x = x_ref[...]
    w = w_ref[...]
    xw = jax.lax.dot_general(x, w, (((1,), (0,)), ((), ())),
                             preferred_element_type=jnp.float32)
    xsq = xsq_ref[...]
    wsq = wsq_ref[...]
    d = (xsq - 2.0 * xw) + wsq

    v = -d
    lane = jax.lax.broadcasted_iota(jnp.int32, (_TB, _NE), 1)
    big = jnp.int32(_NE)
    _C = _NE // 4
    mvs, mis = [], []
    for c in range(4):
        vc = v[:, c * _C:(c + 1) * _C]
        lc = lane[:, c * _C:(c + 1) * _C]
        mv = jnp.max(vc, axis=1, keepdims=True)
        mi = jnp.min(jnp.where(vc == mv, lc, big), axis=1, keepdims=True)
        mvs.append(mv)
        mis.append(mi)
    mcur = mvs[0].astype(jnp.bfloat16).astype(jnp.float32)
    icur = mis[0]
    for c in range(1, 4):
        take = mvs[c] > mcur
        icur = jnp.where(take, mis[c], icur)
        mcur = jnp.where(take, mvs[c], mcur).astype(jnp.bfloat16).astype(jnp.float32)
    idx = icur.astype(jnp.int32)
    idx_ref[...] = idx

    onehot = (lane == idx).astype(jnp.float32)
    q = jax.lax.dot_general(onehot, w, (((1,), (1,)), ((), ())),
                            preferred_element_type=jnp.float32,
                            precision=jax.lax.Precision.HIGHEST)
    qst_ref[...] = x + (q - x)

    dwi = jax.lax.dot_general(x, onehot, (((0,), (0,)), ((), ())),
                              preferred_element_type=jnp.float32,
                              precision=jax.lax.Precision.HIGHEST)
    cnti = jnp.sum(onehot, axis=0, keepdims=True)
    lsum = jnp.sum(jnp.square(q - x))

    @pl.when(i == 0)
    def _init():
        cnt_ref[...] = jnp.zeros_like(cnt_ref)
        dw_ref[...] = jnp.zeros_like(dw_ref)
        loss_ref[...] = jnp.zeros_like(loss_ref)

    cnt_ref[...] += cnti
    dw_ref[...] += dwi
    loss_ref[...] += jnp.full((1, 1), 0.0, jnp.float32) + lsum

    @pl.when(i == _NBLK - 1)
    def _fin():
        c = cnt_ref[...]
        p = c / jnp.float32(_NTOK)
        ent = jnp.sum(p * jnp.log(p + 1e-10))
        ppl_ref[...] = jnp.exp(-ent).reshape(1, 1)
        loss_ref[...] = loss_ref[...] / jnp.float32(_NTOK * _EMB)


def kernel(input, w):
    x = input.reshape(_NTOK, _EMB)
    xsq = jnp.sum(jnp.power(x, 2), axis=1, keepdims=True)
    wsq = jnp.sum(jnp.power(w, 2), axis=0, keepdims=True)
    out_shapes = (
        jax.ShapeDtypeStruct((_NTOK, 1), jnp.int32),
        jax.ShapeDtypeStruct((_NTOK, _EMB), jnp.float32),
        jax.ShapeDtypeStruct((1, _NE), jnp.float32),
        jax.ShapeDtypeStruct((_EMB, _NE), jnp.float32),
        jax.ShapeDtypeStruct((1, 1), jnp.float32),
        jax.ShapeDtypeStruct((1, 1), jnp.float32),
    )
    idx, qst, cnt, dw, loss, ppl = pl.pallas_call(
        _vq_body,
        grid=(_NBLK,),
        in_specs=[
            pl.BlockSpec((_TB, _EMB), lambda i: (i, 0)),
            pl.BlockSpec((_EMB, _NE), lambda i: (0, 0)),
            pl.BlockSpec((_TB, 1), lambda i: (i, 0)),
            pl.BlockSpec((1, _NE), lambda i: (0, 0)),
        ],
        out_specs=[
            pl.BlockSpec((_TB, 1), lambda i: (i, 0)),
            pl.BlockSpec((_TB, _EMB), lambda i: (i, 0)),
            pl.BlockSpec((1, _NE), lambda i: (0, 0)),
            pl.BlockSpec((_EMB, _NE), lambda i: (0, 0)),
            pl.BlockSpec((1, 1), lambda i: (0, 0)),
            pl.BlockSpec((1, 1), lambda i: (0, 0)),
        ],
        out_shape=out_shapes,
    )(x, w, xsq, wsq)
    indices = idx.reshape(input.shape[:-1])
    return (loss.reshape(()), qst.reshape(input.shape), cnt.reshape(1, _NE),
            dw.reshape(1, _EMB, _NE), indices, ppl.reshape(()))


# speedup vs baseline: 1.5839x; 1.5839x over previous
import functools

import jax
import jax.numpy as jnp
from jax import lax
from jax.experimental import pallas as pl
from jax.experimental.pallas import tpu as pltpu
from jax.experimental.pallas import tpu_sc as plsc

_EMB = 32
_NE = 8192
_NTOK = 8192
_TB = 256
_NBLK = _NTOK // _TB
_NW = 32
_BPW = _NTOK // _NW


def _tc1_body(x_ref, w_ref, xsq_ref, wsq_ref, idx_ref, cnt_ref, dw_ref,
              loss_ref, ppl_ref):
    i = pl.program_id(0)
    x = x_ref[...]
    w = w_ref[...]
    xw = jax.lax.dot_general(x, w, (((1,), (0,)), ((), ())),
                             preferred_element_type=jnp.float32)
    d = (xsq_ref[...] - 2.0 * xw) + wsq_ref[...]

    lane = jax.lax.broadcasted_iota(jnp.int32, (_TB, _NE), 1)
    big = jnp.int32(_NE)
    _C = _NE // 4
    mvs, mis = [], []
    for c in range(4):
        dc = d[:, c * _C:(c + 1) * _C]
        lc = lane[:, c * _C:(c + 1) * _C]
        mv = jnp.min(dc, axis=1, keepdims=True)
        mi = jnp.min(jnp.where(dc == mv, lc, big), axis=1, keepdims=True)
        mvs.append(mv)
        mis.append(mi)
    mcur = mvs[0].astype(jnp.bfloat16).astype(jnp.float32)
    icur = mis[0]
    vsel = mvs[0]
    for c in range(1, 4):
        take = mvs[c] < mcur
        icur = jnp.where(take, mis[c], icur)
        vsel = jnp.where(take, mvs[c], vsel)
        mcur = jnp.where(take, mvs[c], mcur).astype(jnp.bfloat16).astype(jnp.float32)
    idx_ref[...] = icur.astype(jnp.int32)

    onehot = (lane == icur).astype(jnp.float32)
    dwi = jax.lax.dot_general(x, onehot, (((0,), (0,)), ((), ())),
                              preferred_element_type=jnp.float32,
                              precision=jax.lax.Precision.HIGHEST)
    cnti = jnp.sum(onehot, axis=0, keepdims=True)

    @pl.when(i == 0)
    def _init():
        cnt_ref[...] = jnp.zeros_like(cnt_ref)
        dw_ref[...] = jnp.zeros_like(dw_ref)
        loss_ref[...] = jnp.zeros_like(loss_ref)

    cnt_ref[...] += cnti
    dw_ref[...] += dwi
    loss_ref[...] += jnp.full((1, 1), 0.0, jnp.float32) + jnp.sum(vsel)

    @pl.when(i == _NBLK - 1)
    def _fin():
        c = cnt_ref[...]
        p = c / jnp.float32(_NTOK)
        ent = jnp.sum(p * jnp.log(p + 1e-10))
        ppl_ref[...] = jnp.exp(-ent).reshape(1, 1)
        loss_ref[...] = loss_ref[...] / jnp.float32(_NTOK * _EMB)


def _sc_gather(wt, idx2):
    mesh = plsc.VectorSubcoreMesh(core_axis_name="c", subcore_axis_name="s")

    @functools.partial(
        pl.kernel, mesh=mesh,
        out_type=(jax.ShapeDtypeStruct((_NTOK, 128), jnp.float32),),
        scratch_types=[pltpu.VMEM((2, 128), jnp.int32),
                       pltpu.VMEM((_BPW, 128), jnp.float32),
                       pltpu.SemaphoreType.DMA],
    )
    def kg(wt_hbm, idx2_hbm, qst_hbm, idx_v, q_v, sem):
        cid = lax.axis_index("c")
        sid = lax.axis_index("s")
        wid = sid * 2 + cid
        base = wid * _BPW
        pltpu.sync_copy(idx2_hbm.at[pl.ds(wid * 2, 2)], idx_v)
        for j in range(2):
            pltpu.async_copy(wt_hbm.at[idx_v.at[j]],
                             q_v.at[pl.ds(j * 128, 128)], sem).wait()
        pltpu.sync_copy(q_v, qst_hbm.at[pl.ds(base, _BPW)])

    qst_pad, = kg(wt, idx2)
    return qst_pad


def kernel(input, w):
    x = input.reshape(_NTOK, _EMB)
    xsq = jnp.sum(jnp.power(x, 2), axis=1, keepdims=True)
    wsq = jnp.sum(jnp.power(w, 2), axis=0, keepdims=True)
    wt = jnp.pad(w.T, ((0, 0), (0, 128 - _EMB)))

    idx, cnt, dw, loss, ppl = pl.pallas_call(
        _tc1_body,
        grid=(_NBLK,),
        in_specs=[
            pl.BlockSpec((_TB, _EMB), lambda i: (i, 0)),
            pl.BlockSpec((_EMB, _NE), lambda i: (0, 0)),
            pl.BlockSpec((_TB, 1), lambda i: (i, 0)),
            pl.BlockSpec((1, _NE), lambda i: (0, 0)),
        ],
        out_specs=[
            pl.BlockSpec((_TB, 1), lambda i: (i, 0)),
            pl.BlockSpec((1, _NE), lambda i: (0, 0)),
            pl.BlockSpec((_EMB, _NE), lambda i: (0, 0)),
            pl.BlockSpec((1, 1), lambda i: (0, 0)),
            pl.BlockSpec((1, 1), lambda i: (0, 0)),
        ],
        out_shape=(
            jax.ShapeDtypeStruct((_NTOK, 1), jnp.int32),
            jax.ShapeDtypeStruct((1, _NE), jnp.float32),
            jax.ShapeDtypeStruct((_EMB, _NE), jnp.float32),
            jax.ShapeDtypeStruct((1, 1), jnp.float32),
            jax.ShapeDtypeStruct((1, 1), jnp.float32),
        ),
    )(x, w, xsq, wsq)

    idx_flat = idx.reshape(_NTOK)
    idx2 = idx_flat.reshape(_NTOK // 128, 128)
    qst_pad = _sc_gather(wt, idx2)
    qst = qst_pad[:, :_EMB]

    return (loss.reshape(()), qst.reshape(input.shape), cnt,
            dw.reshape(1, _EMB, _NE), idx_flat.reshape(input.shape[:-1]),
            ppl.reshape(()))


# speedup vs baseline: 2.5304x; 1.5976x over previous
import functools

import jax
import jax.numpy as jnp
from jax import lax
from jax.experimental import pallas as pl
from jax.experimental.pallas import tpu as pltpu
from jax.experimental.pallas import tpu_sc as plsc

_EMB = 32
_NE = 8192
_NTOK = 8192
_TB = 256
_NBLK = _NTOK // _TB
_NW = 32
_BPW = _NTOK // _NW


def _tc1_body(x_ref, w_ref, xsq_ref, wsq_ref, idx_ref, cnt_ref, dw_ref,
              loss_ref, ppl_ref):
    i = pl.program_id(0)
    x = x_ref[...]
    w = w_ref[...]
    xw = jax.lax.dot_general(x, w, (((1,), (0,)), ((), ())),
                             preferred_element_type=jnp.float32)
    d = (xsq_ref[...] - 2.0 * xw) + wsq_ref[...]

    lane = jax.lax.broadcasted_iota(jnp.int32, (_TB, _NE), 1)
    big = jnp.int32(_NE)
    _C = _NE // 4
    mvs, mis = [], []
    for c in range(4):
        dc = d[:, c * _C:(c + 1) * _C]
        lc = lane[:, c * _C:(c + 1) * _C]
        mv = jnp.min(dc, axis=1, keepdims=True)
        mi = jnp.min(jnp.where(dc == mv, lc, big), axis=1, keepdims=True)
        mvs.append(mv)
        mis.append(mi)
    mcur = mvs[0].astype(jnp.bfloat16).astype(jnp.float32)
    icur = mis[0]
    vsel = mvs[0]
    for c in range(1, 4):
        take = mvs[c] < mcur
        icur = jnp.where(take, mis[c], icur)
        vsel = jnp.where(take, mvs[c], vsel)
        mcur = jnp.where(take, mvs[c], mcur).astype(jnp.bfloat16).astype(jnp.float32)
    idx_ref[...] = icur.astype(jnp.int32)

    onehot = (lane == icur).astype(jnp.bfloat16)
    dwi = jax.lax.dot_general(x.astype(jnp.bfloat16), onehot,
                              (((0,), (0,)), ((), ())),
                              preferred_element_type=jnp.float32)
    cnti = jax.lax.dot_general(jnp.ones((1, _TB), jnp.bfloat16), onehot,
                               (((1,), (0,)), ((), ())),
                               preferred_element_type=jnp.float32)

    @pl.when(i == 0)
    def _init():
        cnt_ref[...] = jnp.zeros_like(cnt_ref)
        dw_ref[...] = jnp.zeros_like(dw_ref)
        loss_ref[...] = jnp.zeros_like(loss_ref)

    cnt_ref[...] += cnti
    dw_ref[...] += dwi
    loss_ref[...] += jnp.full((1, 1), 0.0, jnp.float32) + jnp.sum(vsel)

    @pl.when(i == _NBLK - 1)
    def _fin():
        c = cnt_ref[...]
        p = c / jnp.float32(_NTOK)
        ent = jnp.sum(p * jnp.log(p + 1e-10))
        ppl_ref[...] = jnp.exp(-ent).reshape(1, 1)
        loss_ref[...] = loss_ref[...] / jnp.float32(_NTOK * _EMB)


def _sc_gather(wt, idx2):
    mesh = plsc.VectorSubcoreMesh(core_axis_name="c", subcore_axis_name="s")

    @functools.partial(
        pl.kernel, mesh=mesh,
        out_type=(jax.ShapeDtypeStruct((_NTOK, 128), jnp.float32),),
        scratch_types=[pltpu.VMEM((2, 128), jnp.int32),
                       pltpu.VMEM((_BPW, 128), jnp.float32),
                       pltpu.SemaphoreType.DMA],
    )
    def kg(wt_hbm, idx2_hbm, qst_hbm, idx_v, q_v, sem):
        cid = lax.axis_index("c")
        sid = lax.axis_index("s")
        wid = sid * 2 + cid
        base = wid * _BPW
        pltpu.sync_copy(idx2_hbm.at[pl.ds(wid * 2, 2)], idx_v)
        for j in range(2):
            pltpu.async_copy(wt_hbm.at[idx_v.at[j]],
                             q_v.at[pl.ds(j * 128, 128)], sem).wait()
        pltpu.sync_copy(q_v, qst_hbm.at[pl.ds(base, _BPW)])

    qst_pad, = kg(wt, idx2)
    return qst_pad


def kernel(input, w):
    x = input.reshape(_NTOK, _EMB)
    xsq = jnp.sum(jnp.power(x, 2), axis=1, keepdims=True)
    wsq = jnp.sum(jnp.power(w, 2), axis=0, keepdims=True)
    wt = jnp.pad(w.T, ((0, 0), (0, 128 - _EMB)))

    idx, cnt, dw, loss, ppl = pl.pallas_call(
        _tc1_body,
        grid=(_NBLK,),
        in_specs=[
            pl.BlockSpec((_TB, _EMB), lambda i: (i, 0)),
            pl.BlockSpec((_EMB, _NE), lambda i: (0, 0)),
            pl.BlockSpec((_TB, 1), lambda i: (i, 0)),
            pl.BlockSpec((1, _NE), lambda i: (0, 0)),
        ],
        out_specs=[
            pl.BlockSpec((_TB, 1), lambda i: (i, 0)),
            pl.BlockSpec((1, _NE), lambda i: (0, 0)),
            pl.BlockSpec((_EMB, _NE), lambda i: (0, 0)),
            pl.BlockSpec((1, 1), lambda i: (0, 0)),
            pl.BlockSpec((1, 1), lambda i: (0, 0)),
        ],
        out_shape=(
            jax.ShapeDtypeStruct((_NTOK, 1), jnp.int32),
            jax.ShapeDtypeStruct((1, _NE), jnp.float32),
            jax.ShapeDtypeStruct((_EMB, _NE), jnp.float32),
            jax.ShapeDtypeStruct((1, 1), jnp.float32),
            jax.ShapeDtypeStruct((1, 1), jnp.float32),
        ),
    )(x, w, xsq, wsq)

    idx_flat = idx.reshape(_NTOK)
    idx2 = idx_flat.reshape(_NTOK // 128, 128)
    qst_pad = _sc_gather(wt, idx2)
    qst = qst_pad[:, :_EMB]

    return (loss.reshape(()), qst.reshape(input.shape), cnt,
            dw.reshape(1, _EMB, _NE), idx_flat.reshape(input.shape[:-1]),
            ppl.reshape(()))
